# Initial kernel scaffold; baseline (speedup 1.0000x reference)
import jax
import jax.numpy as jnp
from jax.experimental import pallas as pl


def kernel(y_pred0, y_pred1, y_true0, y_true1, batch_idx, cls, bboxes):
    raise NotImplementedError("write your pallas kernel here")


# speedup vs baseline: 4.1736x; 4.1736x over previous
import functools

import jax
import jax.numpy as jnp
from jax.experimental import pallas as pl
from jax.experimental.pallas import tpu as pltpu


def _mask_kernel(bid_ref, box0_ref, box1_ref, m0_ref, s0_ref, m1_ref, s1_ref):
    for S, m_ref, box_ref, s_ref in (
        (80, m0_ref, box0_ref, s0_ref),
        (40, m1_ref, box1_ref, s1_ref),
    ):
        m_ref[...] = jnp.zeros_like(m_ref)
        xi = jax.lax.broadcasted_iota(jnp.int32, (1, S, S), 2)
        yi = jax.lax.broadcasted_iota(jnp.int32, (1, S, S), 1)
        xf = xi.astype(jnp.float32)
        yf = yi.astype(jnp.float32)

        def body(i, carry, S=S, m_ref=m_ref, box_ref=box_ref, xi=xi, yi=yi,
                 xf=xf, yf=yf):
            b = bid_ref[i]
            xc = box_ref[i, 0]
            yc = box_ref[i, 1]
            wd = box_ref[i, 2]
            ht = box_ref[i, 3]
            xl = jnp.maximum(xc - wd // 2, 0)
            yt = jnp.maximum(yc - ht // 2, 0)
            xr = jnp.minimum(xc + wd // 2, S - 1)
            yd = jnp.minimum(yc + ht // 2, S - 1)
            w = (xr - xl + 1).astype(jnp.float32)
            h = (yd - yt + 1).astype(jnp.float32)
            xcf = xc.astype(jnp.float32)
            ycf = yc.astype(jnp.float32)
            g = jnp.exp(-((xf - xcf) ** 2 / (w * w) + (yf - ycf) ** 2 / (h * h)))
            inside = (xi >= xl) & (xi <= xr) & (yi >= yt) & (yi <= yd)
            cand = jnp.where(inside, g, 0.0)
            m_ref[pl.ds(b, 1)] = jnp.maximum(m_ref[pl.ds(b, 1)], cand)
            return carry

        jax.lax.fori_loop(0, bid_ref.shape[0], body, 0)
        s_ref[0, 0] = jnp.sum(m_ref[...])


def _masked_red_kernel(p_ref, t_ref, m_ref, out_ref):
    b = pl.program_id(0)
    c = pl.program_id(1)
    d = p_ref[...] - t_ref[...]
    dsum = jnp.sum(d * d, axis=1)
    m = m_ref[...]
    val = jnp.sum(m * m * dsum)

    @pl.when((b == 0) & (c == 0))
    def _():
        out_ref[0, 0] = 0.0

    out_ref[0, 0] += val


def _masks(batch_idx, bboxes):
    bid = batch_idx.astype(jnp.int32)
    box0 = jnp.floor(bboxes * 80.0).astype(jnp.int32)
    box1 = jnp.floor(bboxes * 40.0).astype(jnp.int32)
    grid_spec = pltpu.PrefetchScalarGridSpec(
        num_scalar_prefetch=3,
        grid=(1,),
        in_specs=[],
        out_specs=[
            pl.BlockSpec((8, 80, 80), lambda i, *_: (0, 0, 0)),
            pl.BlockSpec(memory_space=pltpu.SMEM),
            pl.BlockSpec((8, 40, 40), lambda i, *_: (0, 0, 0)),
            pl.BlockSpec(memory_space=pltpu.SMEM),
        ],
    )
    return pl.pallas_call(
        _mask_kernel,
        grid_spec=grid_spec,
        out_shape=[
            jax.ShapeDtypeStruct((8, 80, 80), jnp.float32),
            jax.ShapeDtypeStruct((1, 1), jnp.float32),
            jax.ShapeDtypeStruct((8, 40, 40), jnp.float32),
            jax.ShapeDtypeStruct((1, 1), jnp.float32),
        ],
    )(bid, box0, box1)


def _masked_red(p, t, m, cb):
    B, C, S, _ = p.shape
    grid = (B, C // cb)
    out = pl.pallas_call(
        _masked_red_kernel,
        grid=grid,
        in_specs=[
            pl.BlockSpec((1, cb, S, S), lambda b, c: (b, c, 0, 0)),
            pl.BlockSpec((1, cb, S, S), lambda b, c: (b, c, 0, 0)),
            pl.BlockSpec((1, S, S), lambda b, c: (b, 0, 0)),
        ],
        out_specs=pl.BlockSpec(memory_space=pltpu.SMEM),
        out_shape=jax.ShapeDtypeStruct((1, 1), jnp.float32),
    )(p, t, m)
    return out[0, 0]


@jax.jit
def kernel(y_pred0, y_pred1, y_true0, y_true1, batch_idx, cls, bboxes):
    m0, s0, m1, s1 = _masks(batch_idx, bboxes)
    r0 = _masked_red(y_pred0, y_true0, m0, 32)
    r1 = _masked_red(y_pred1, y_true1, m1, 128)
    n0 = 256.0 * s0[0, 0]
    n1 = 256.0 * s1[0, 0]
    return (r0 / n0 + r1 / n1) * 0.5


# speedup vs baseline: 6.1350x; 1.4700x over previous
import functools

import jax
import jax.numpy as jnp
from jax.experimental import pallas as pl
from jax.experimental.pallas import tpu as pltpu


def _mask_kernel(bid_ref, box0_ref, box1_ref, m0_ref, s0_ref, m1_ref, s1_ref):
    for S, m_ref, box_ref, s_ref in (
        (80, m0_ref, box0_ref, s0_ref),
        (40, m1_ref, box1_ref, s1_ref),
    ):
        m_ref[...] = jnp.zeros_like(m_ref)
        xi = jax.lax.broadcasted_iota(jnp.int32, (1, S, S), 2)
        yi = jax.lax.broadcasted_iota(jnp.int32, (1, S, S), 1)
        xf = xi.astype(jnp.float32)
        yf = yi.astype(jnp.float32)

        def body(i, carry, S=S, m_ref=m_ref, box_ref=box_ref, xi=xi, yi=yi,
                 xf=xf, yf=yf):
            b = bid_ref[i]
            xc = box_ref[i, 0]
            yc = box_ref[i, 1]
            wd = box_ref[i, 2]
            ht = box_ref[i, 3]
            xl = jnp.maximum(xc - wd // 2, 0)
            yt = jnp.maximum(yc - ht // 2, 0)
            xr = jnp.minimum(xc + wd // 2, S - 1)
            yd = jnp.minimum(yc + ht // 2, S - 1)
            w = (xr - xl + 1).astype(jnp.float32)
            h = (yd - yt + 1).astype(jnp.float32)
            xcf = xc.astype(jnp.float32)
            ycf = yc.astype(jnp.float32)
            g = jnp.exp(-((xf - xcf) ** 2 / (w * w) + (yf - ycf) ** 2 / (h * h)))
            inside = (xi >= xl) & (xi <= xr) & (yi >= yt) & (yi <= yd)
            cand = jnp.where(inside, g, 0.0)
            m_ref[pl.ds(b, 1)] = jnp.maximum(m_ref[pl.ds(b, 1)], cand)
            return carry

        jax.lax.fori_loop(0, bid_ref.shape[0], body, 0)
        s_ref[0, 0] = jnp.sum(m_ref[...])


def _masked_red_kernel(p_ref, t_ref, m_ref, out_ref):
    b = pl.program_id(0)
    c = pl.program_id(1)
    d = p_ref[...] - t_ref[...]
    dsum = jnp.sum(d * d, axis=1)
    m = m_ref[...]
    val = jnp.sum(m * m * dsum)

    @pl.when((b == 0) & (c == 0))
    def _():
        out_ref[0, 0] = 0.0

    out_ref[0, 0] += val


def _masks(batch_idx, bboxes):
    bid = batch_idx.astype(jnp.int32)
    box0 = jnp.floor(bboxes * 80.0).astype(jnp.int32)
    box1 = jnp.floor(bboxes * 40.0).astype(jnp.int32)
    grid_spec = pltpu.PrefetchScalarGridSpec(
        num_scalar_prefetch=3,
        grid=(1,),
        in_specs=[],
        out_specs=[
            pl.BlockSpec((8, 80, 80), lambda i, *_: (0, 0, 0)),
            pl.BlockSpec(memory_space=pltpu.SMEM),
            pl.BlockSpec((8, 40, 40), lambda i, *_: (0, 0, 0)),
            pl.BlockSpec(memory_space=pltpu.SMEM),
        ],
    )
    return pl.pallas_call(
        _mask_kernel,
        grid_spec=grid_spec,
        out_shape=[
            jax.ShapeDtypeStruct((8, 80, 80), jnp.float32),
            jax.ShapeDtypeStruct((1, 1), jnp.float32),
            jax.ShapeDtypeStruct((8, 40, 40), jnp.float32),
            jax.ShapeDtypeStruct((1, 1), jnp.float32),
        ],
    )(bid, box0, box1)


def _masked_red(p, t, m, cb):
    B, C, S, _ = p.shape
    ss = S * S
    p = p.reshape(B, C, ss)
    t = t.reshape(B, C, ss)
    m = m.reshape(B, 1, ss)
    grid = (B, C // cb)
    out = pl.pallas_call(
        _masked_red_kernel,
        grid=grid,
        in_specs=[
            pl.BlockSpec((1, cb, ss), lambda b, c: (b, c, 0)),
            pl.BlockSpec((1, cb, ss), lambda b, c: (b, c, 0)),
            pl.BlockSpec((1, 1, ss), lambda b, c: (b, 0, 0)),
        ],
        out_specs=pl.BlockSpec(memory_space=pltpu.SMEM),
        out_shape=jax.ShapeDtypeStruct((1, 1), jnp.float32),
    )(p, t, m)
    return out[0, 0]


@jax.jit
def kernel(y_pred0, y_pred1, y_true0, y_true1, batch_idx, cls, bboxes):
    m0, s0, m1, s1 = _masks(batch_idx, bboxes)
    r0 = _masked_red(y_pred0, y_true0, m0, 32)
    r1 = _masked_red(y_pred1, y_true1, m1, 128)
    n0 = 256.0 * s0[0, 0]
    n1 = 256.0 * s1[0, 0]
    return (r0 / n0 + r1 / n1) * 0.5


# speedup vs baseline: 6.4746x; 1.0554x over previous
import functools

import jax
import jax.numpy as jnp
from jax.experimental import pallas as pl
from jax.experimental.pallas import tpu as pltpu


def _mask_kernel(bid_ref, bb_ref, m_ref, *, S):
    ss = S * S
    box = jnp.floor(bb_ref[...] * S).astype(jnp.int32)
    xc = box[:, 0:1]
    yc = box[:, 1:2]
    wd = box[:, 2:3]
    ht = box[:, 3:4]
    xl = jnp.maximum(xc - wd // 2, 0)
    yt = jnp.maximum(yc - ht // 2, 0)
    xr = jnp.minimum(xc + wd // 2, S - 1)
    yd = jnp.minimum(yc + ht // 2, S - 1)
    w = (xr - xl + 1).astype(jnp.float32)
    h = (yd - yt + 1).astype(jnp.float32)
    idx = jax.lax.broadcasted_iota(jnp.int32, (1, ss), 1)
    xflat = idx % S
    yflat = idx // S
    dx = xflat.astype(jnp.float32) - xc.astype(jnp.float32)
    dy = yflat.astype(jnp.float32) - yc.astype(jnp.float32)
    arg = dx * dx / (w * w) + dy * dy / (h * h)
    g = jnp.exp(-arg)
    inside = ((xflat >= xl) & (xflat <= xr)
              & (yflat >= yt) & (yflat <= yd))
    g = jnp.where(inside, g, 0.0)
    bid = bid_ref[...]
    for b in range(8):
        gb = jnp.where(bid == b, g, 0.0)
        m_ref[b] = jnp.max(gb, axis=0, keepdims=True)


def _dsum_kernel(p_ref, t_ref, d_ref):
    c = pl.program_id(1)
    d = p_ref[...] - t_ref[...]
    s = jnp.sum(d * d, axis=1, keepdims=True)

    @pl.when(c == 0)
    def _():
        d_ref[...] = s

    @pl.when(c != 0)
    def _():
        d_ref[...] += s


def _combine_kernel(m0_ref, d0_ref, m1_ref, d1_ref, o_ref):
    acc = jnp.float32(0.0)
    for m_ref, d_ref in ((m0_ref, d0_ref), (m1_ref, d1_ref)):
        m = m_ref[...]
        r = jnp.sum(m * m * d_ref[...])
        sm = jnp.sum(m)
        acc = acc + r / (256.0 * sm)
    o_ref[0, 0] = 0.5 * acc


def _masks(batch_idx, bboxes, S):
    bid = batch_idx.astype(jnp.int32).reshape(64, 1)
    return pl.pallas_call(
        functools.partial(_mask_kernel, S=S),
        out_shape=jax.ShapeDtypeStruct((8, 1, S * S), jnp.float32),
    )(bid, bboxes)


def _dsum(p, t, cb):
    B, C, S, _ = p.shape
    ss = S * S
    p = p.reshape(B, C, ss)
    t = t.reshape(B, C, ss)
    grid = (B, C // cb)
    return pl.pallas_call(
        _dsum_kernel,
        grid=grid,
        in_specs=[
            pl.BlockSpec((1, cb, ss), lambda b, c: (b, c, 0)),
            pl.BlockSpec((1, cb, ss), lambda b, c: (b, c, 0)),
        ],
        out_specs=pl.BlockSpec((1, 1, ss), lambda b, c: (b, 0, 0)),
        out_shape=jax.ShapeDtypeStruct((B, 1, ss), jnp.float32),
    )(p, t)


@jax.jit
def kernel(y_pred0, y_pred1, y_true0, y_true1, batch_idx, cls, bboxes):
    d0 = _dsum(y_pred0, y_true0, 32)
    d1 = _dsum(y_pred1, y_true1, 128)
    m0 = _masks(batch_idx, bboxes, 80)
    m1 = _masks(batch_idx, bboxes, 40)
    out = pl.pallas_call(
        _combine_kernel,
        out_shape=jax.ShapeDtypeStruct((1, 1), jnp.float32),
        out_specs=pl.BlockSpec(memory_space=pltpu.SMEM),
    )(m0, d0, m1, d1)
    return out[0, 0]
